# baseline (speedup 1.0000x reference)
import functools

import jax
import jax.numpy as jnp
from jax.experimental import pallas as pl
from jax.experimental.pallas import tpu as pltpu


def _prep_body(eids_ref, w0_ref, w1_ref, ws_ref):
    ws_ref[...] = (w0_ref[0] + w1_ref[0]).astype(jnp.bfloat16).T


def _mm_body(eids_ref, x_ref, ws_ref, b0_ref, b1_ref, o_ref, xb_ref):
    i = pl.program_id(0)
    n_x = pl.num_programs(0) - 1
    slot = jax.lax.rem(i, 2)

    @pl.when(i < n_x)
    def _cast_ahead():
        xb_ref[slot] = x_ref[...].astype(jnp.bfloat16)

    @pl.when(i > 0)
    def _matmul_behind():
        acc = jax.lax.dot_general(
            xb_ref[1 - slot], ws_ref[...],
            dimension_numbers=(((1,), (0,)), ((), ())),
            preferred_element_type=jnp.float32)
        o_ref[...] = acc + (b0_ref[0, 0] + b1_ref[0, 0])[None, :]


@functools.partial(jax.jit, static_argnames=("bm", "bp"))
def _rmoe(x, W, b, expert_ids, bm, bp):
    B, D = x.shape
    eids = expert_ids.astype(jnp.int32)
    b3 = b.reshape(b.shape[0], 1, b.shape[1])

    prep_spec = pltpu.PrefetchScalarGridSpec(
        num_scalar_prefetch=1,
        grid=(D // bp,),
        in_specs=[
            pl.BlockSpec((1, bp, D), lambda j, eids: (eids[0], j, 0)),
            pl.BlockSpec((1, bp, D), lambda j, eids: (eids[1], j, 0)),
        ],
        out_specs=pl.BlockSpec((D, bp), lambda j, eids: (0, j)),
    )
    ws = pl.pallas_call(
        _prep_body,
        grid_spec=prep_spec,
        out_shape=jax.ShapeDtypeStruct((D, D), jnp.bfloat16),
    )(eids, W, W)

    n_x = B // bm
    mm_spec = pltpu.PrefetchScalarGridSpec(
        num_scalar_prefetch=1,
        grid=(n_x + 1,),
        in_specs=[
            pl.BlockSpec((bm, D), lambda i, eids: (jnp.minimum(i, n_x - 1), 0)),
            pl.BlockSpec((D, D), lambda i, eids: (0, 0)),
            pl.BlockSpec((1, 1, D), lambda i, eids: (eids[0], 0, 0)),
            pl.BlockSpec((1, 1, D), lambda i, eids: (eids[1], 0, 0)),
        ],
        out_specs=pl.BlockSpec(
            (bm, D), lambda i, eids: (jnp.maximum(i - 1, 0), 0)),
        scratch_shapes=[pltpu.VMEM((2, bm, D), jnp.bfloat16)],
    )
    return pl.pallas_call(
        _mm_body,
        grid_spec=mm_spec,
        out_shape=jax.ShapeDtypeStruct((B, D), jnp.float32),
        compiler_params=pltpu.CompilerParams(
            dimension_semantics=("arbitrary",)),
    )(eids, x, ws, b3, b3)


def kernel(x, W, b, expert_ids):
    return _rmoe(x, W, b, expert_ids, bm=1024, bp=1024)


# speedup vs baseline: 1.0509x; 1.0509x over previous
import functools

import jax
import jax.numpy as jnp
from jax.experimental import pallas as pl
from jax.experimental.pallas import tpu as pltpu


def _prep_body(eids_ref, w0_ref, w1_ref, ws_ref):
    ws_ref[...] = (w0_ref[0] + w1_ref[0]).astype(jnp.bfloat16).T


def _mm_body(eids_ref, x_ref, ws_ref, b0_ref, b1_ref, o_ref, xb_ref):
    xb_ref[...] = x_ref[...].astype(jnp.bfloat16)
    acc = jax.lax.dot_general(
        xb_ref[...], ws_ref[...],
        dimension_numbers=(((1,), (0,)), ((), ())),
        preferred_element_type=jnp.float32)
    o_ref[...] = acc + (b0_ref[0, 0] + b1_ref[0, 0])[None, :]


@functools.partial(jax.jit, static_argnames=("bm", "bp"))
def _rmoe(x, W, b, expert_ids, bm, bp):
    B, D = x.shape
    eids = expert_ids.astype(jnp.int32)
    b3 = b.reshape(b.shape[0], 1, b.shape[1])

    prep_spec = pltpu.PrefetchScalarGridSpec(
        num_scalar_prefetch=1,
        grid=(D // bp,),
        in_specs=[
            pl.BlockSpec((1, bp, D), lambda j, eids: (eids[0], j, 0)),
            pl.BlockSpec((1, bp, D), lambda j, eids: (eids[1], j, 0)),
        ],
        out_specs=pl.BlockSpec((D, bp), lambda j, eids: (0, j)),
    )
    ws = pl.pallas_call(
        _prep_body,
        grid_spec=prep_spec,
        out_shape=jax.ShapeDtypeStruct((D, D), jnp.bfloat16),
    )(eids, W, W)

    mm_spec = pltpu.PrefetchScalarGridSpec(
        num_scalar_prefetch=1,
        grid=(B // bm,),
        in_specs=[
            pl.BlockSpec((bm, D), lambda i, eids: (i, 0)),
            pl.BlockSpec((D, D), lambda i, eids: (0, 0)),
            pl.BlockSpec((1, 1, D), lambda i, eids: (eids[0], 0, 0)),
            pl.BlockSpec((1, 1, D), lambda i, eids: (eids[1], 0, 0)),
        ],
        out_specs=pl.BlockSpec((bm, D), lambda i, eids: (i, 0)),
        scratch_shapes=[pltpu.VMEM((bm, D), jnp.bfloat16)],
    )
    return pl.pallas_call(
        _mm_body,
        grid_spec=mm_spec,
        out_shape=jax.ShapeDtypeStruct((B, D), jnp.float32),
        compiler_params=pltpu.CompilerParams(
            dimension_semantics=("parallel",),
            vmem_limit_bytes=110 * 1024 * 1024),
    )(eids, x, ws, b3, b3)


def kernel(x, W, b, expert_ids):
    return _rmoe(x, W, b, expert_ids, bm=1024, bp=1024)
